# baseline (speedup 1.0000x reference)
import functools

import jax
import jax.numpy as jnp
from jax import lax
from jax.experimental import pallas as pl
from jax.experimental.pallas import tpu as pltpu
from jax.experimental.pallas import tpu_sc as plsc

NUM_EMBEDDINGS = 1000000
FEATURES = 32
BATCH = 4096
LENGTH = 200

NC = 2
NS = 16
NW = NC * NS

FT = 4
FS = 8
BT = 32
BL = 128

SB = 8
BTS = BT // SB
CHUNK = BTS * BL
NBLK = LENGTH * SB
BLK_PER_W = NBLK // NW
NG = CHUNK // 16


def _make_gather():
    mesh = plsc.VectorSubcoreMesh(
        core_axis_name="c", subcore_axis_name="s", num_cores=NC, num_subcores=NS
    )

    @functools.partial(
        pl.kernel,
        out_type=jax.ShapeDtypeStruct((LENGTH, FT, BT, FS, BL), jnp.float32),
        mesh=mesh,
        compiler_params=pltpu.CompilerParams(
            use_tc_tiling_on_sc=False,
            needs_layout_passes=False,
            disable_bounds_checks=True,
        ),
        scratch_types=[
            pltpu.VMEM((CHUNK,), jnp.int32),
            pltpu.VMEM((CHUNK,), jnp.int32),
            pltpu.VMEM((CHUNK, FEATURES), jnp.float32),
            pltpu.VMEM((CHUNK, FEATURES), jnp.float32),
            pltpu.VMEM((FT, BTS, FS, BL), jnp.float32),
            pltpu.SemaphoreType.DMA,
            pltpu.SemaphoreType.DMA,
            pltpu.SemaphoreType.DMA,
            pltpu.SemaphoreType.DMA,
            pltpu.SemaphoreType.DMA,
        ],
    )
    def k(table_hbm, idx_hbm, out_hbm, *refs):
        idx_v = refs[0:2]
        rows_v = refs[2:4]
        tbuf_v = refs[4]
        isem = refs[5:7]
        gsem = refs[7:9]
        wsem = refs[9]

        wid = lax.axis_index("s") * NC + lax.axis_index("c")
        base = wid * BLK_PER_W
        lane = jnp.arange(16, dtype=jnp.int32)

        def start_idx(m, b):
            l = m // SB
            sb = m % SB
            return pltpu.async_copy(
                idx_hbm.at[l, pl.ds(sb * CHUNK, CHUNK)], idx_v[b], isem[b]
            )

        def start_gather(b):
            return pltpu.async_copy(table_hbm.at[idx_v[b]], rows_v[b], gsem[b])

        def transpose(b):
            for g in range(NG):
                rows16 = g * 16 + lane
                btr, blo = g // 8, (g % 8) * 16
                for f in range(FEATURES):
                    v = plsc.load_gather(
                        rows_v[b],
                        [rows16, jnp.full((16,), f, dtype=jnp.int32)],
                    )
                    tbuf_v[f // FS, btr, f % FS, pl.ds(blo, 16)] = v

        def write_out(m):
            l = m // SB
            sb = m % SB
            descs = [
                pltpu.async_copy(
                    tbuf_v.at[ft],
                    out_hbm.at[l, ft, pl.ds(sb * BTS, BTS)],
                    wsem,
                )
                for ft in range(FT)
            ]
            for d in descs:
                d.wait()

        start_idx(base, 0).wait()
        start_idx(base + 1, 1)
        start_gather(0)

        def pair_body(p, carry):
            for q in range(2):
                m = base + 2 * p + q
                nq = 1 - q
                pltpu.make_async_copy(
                    table_hbm.at[idx_v[q]], rows_v[q], gsem[q]
                ).wait()
                @pl.when(2 * p + q + 2 < BLK_PER_W)
                def _():
                    start_idx(m + 2, q)

                @pl.when(2 * p + q + 1 < BLK_PER_W)
                def _():
                    pltpu.make_async_copy(
                        idx_hbm.at[0, pl.ds(0, CHUNK)], idx_v[nq], isem[nq]
                    ).wait()
                    start_gather(nq)

                transpose(q)
                write_out(m)
            return carry

        lax.fori_loop(0, BLK_PER_W // 2, pair_body, 0)

    return k


_gather = _make_gather()


def kernel(inputs, embedding):
    idx = jnp.transpose(inputs)
    out5 = _gather(embedding, idx)
    return out5.transpose(2, 4, 0, 1, 3).reshape(BATCH, LENGTH, FEATURES)


# speedup vs baseline: 2.0920x; 2.0920x over previous
import functools

import jax
import jax.numpy as jnp
from jax import lax
from jax.experimental import pallas as pl
from jax.experimental.pallas import tpu as pltpu
from jax.experimental.pallas import tpu_sc as plsc

NUM_EMBEDDINGS = 1000000
FEATURES = 32
BATCH = 4096
LENGTH = 200

NC = 2
NS = 16
NW = NC * NS

FT = 4
FS = 8
BT = 32
BL = 128

SB = 8
BTS = BT // SB
CHUNK = BTS * BL
NBLK = LENGTH * SB
BLK_PER_W = NBLK // NW

PBL = BL + 1
FTS = BTS * FS * PBL
SJS = FS * PBL
TBUF = FT * FTS


def _make_gather():
    mesh = plsc.VectorSubcoreMesh(
        core_axis_name="c", subcore_axis_name="s", num_cores=NC, num_subcores=NS
    )

    @functools.partial(
        pl.kernel,
        out_type=jax.ShapeDtypeStruct((LENGTH, FT, BT, FS, BL), jnp.float32),
        mesh=mesh,
        compiler_params=pltpu.CompilerParams(
            use_tc_tiling_on_sc=False,
            needs_layout_passes=False,
            disable_bounds_checks=True,
        ),
        scratch_types=[
            pltpu.VMEM((CHUNK,), jnp.int32),
            pltpu.VMEM((CHUNK,), jnp.int32),
            pltpu.VMEM((CHUNK, FEATURES), jnp.float32),
            pltpu.VMEM((CHUNK, FEATURES), jnp.float32),
            pltpu.VMEM((FT, BTS, FS, PBL), jnp.float32),
            pltpu.VMEM((FT, BTS, FS, PBL), jnp.float32),
            pltpu.SemaphoreType.DMA,
            pltpu.SemaphoreType.DMA,
            pltpu.SemaphoreType.DMA,
            pltpu.SemaphoreType.DMA,
            pltpu.SemaphoreType.DMA,
            pltpu.SemaphoreType.DMA,
        ],
    )
    def k(table_hbm, idx_hbm, out_hbm, *refs):
        idx_v = refs[0:2]
        rows_v = refs[2:4]
        tbuf_v = refs[4:6]
        isem = refs[6:8]
        gsem = refs[8:10]
        wsem = refs[10:12]

        wid = lax.axis_index("s") * NC + lax.axis_index("c")
        base = wid * BLK_PER_W
        lane = jnp.arange(16, dtype=jnp.int32)
        ftv = lane // FS
        fsv = lane % FS

        def start_idx(m, b):
            l = m // SB
            sb = m % SB
            return pltpu.async_copy(
                idx_hbm.at[l, pl.ds(sb * CHUNK, CHUNK)], idx_v[b], isem[b]
            )

        def start_gather(b):
            return pltpu.async_copy(table_hbm.at[idx_v[b]], rows_v[b], gsem[b])

        def drain_writes(b):
            for _ in range(FT):
                pltpu.make_async_copy(
                    tbuf_v[b].at[0, :, :, pl.ds(0, BL)],
                    out_hbm.at[0, 0, pl.ds(0, BTS)],
                    wsem[b],
                ).wait()

        def transpose(b):
            rv = rows_v[b]
            tb = tbuf_v[b]

            def tr_body(g, carry):
                btrv = jnp.full((16,), g // 16, dtype=jnp.int32)
                blbase = (g % 16) * 8
                vals = []
                blvs = []
                for u in range(8):
                    j = g * 8 + u
                    blvs.append(jnp.full((16,), blbase + u, dtype=jnp.int32))
                    vals.append((rv[j, pl.ds(0, 16)], rv[j, pl.ds(16, 16)]))
                for (vlo, vhi), blv in zip(vals, blvs):
                    plsc.store_scatter(tb, [ftv, btrv, fsv, blv], vlo)
                    plsc.store_scatter(tb, [ftv + 2, btrv, fsv, blv], vhi)
                return carry

            lax.fori_loop(0, CHUNK // 8, tr_body, 0)

        def write_out(m, b):
            l = m // SB
            sb = m % SB
            for ft in range(FT):
                pltpu.async_copy(
                    tbuf_v[b].at[ft, :, :, pl.ds(0, BL)],
                    out_hbm.at[l, ft, pl.ds(sb * BTS, BTS)],
                    wsem[b],
                )

        start_idx(base, 0).wait()
        start_idx(base + 1, 1)
        start_gather(0)

        def pair_body(p, carry):
            for q in range(2):
                m = base + 2 * p + q
                nq = 1 - q
                pltpu.make_async_copy(
                    table_hbm.at[idx_v[q]], rows_v[q], gsem[q]
                ).wait()
                @pl.when(2 * p + q + 2 < BLK_PER_W)
                def _():
                    start_idx(m + 2, q)

                @pl.when(2 * p + q + 1 < BLK_PER_W)
                def _():
                    pltpu.make_async_copy(
                        idx_hbm.at[0, pl.ds(0, CHUNK)], idx_v[nq], isem[nq]
                    ).wait()
                    start_gather(nq)

                @pl.when(2 * p + q >= 2)
                def _():
                    drain_writes(q)

                transpose(q)
                write_out(m, q)
            return carry

        lax.fori_loop(0, BLK_PER_W // 2, pair_body, 0)
        drain_writes(0)
        drain_writes(1)

    return k


_gather = _make_gather()


def kernel(inputs, embedding):
    idx = jnp.transpose(inputs)
    out5 = _gather(embedding, idx)
    return out5.transpose(2, 4, 0, 1, 3).reshape(BATCH, LENGTH, FEATURES)
